# baseline (device time: 37042 ns/iter reference)
import jax
import jax.numpy as jnp
from jax import lax
from jax.experimental import pallas as pl
from jax.experimental.pallas import tpu as pltpu

N_DEV = 4
HQ_PER = 8
DH = 64
BLK = 64
SQ = 512
QROWS = 256
D = 768

BF16 = jnp.bfloat16
F32 = jnp.float32


def _body(x_ref, wq_ref, k_hbm, v_hbm, wo_ref, out_ref, ctx_scr,
          k_ref, v_ref, snd1, rcv1, snd2, rcv2, sndq, rcvq, rcvf,
          s_sems, r_sems, cp_sems):
    my = lax.axis_index("i")

    row0 = my * HQ_PER * DH
    kv_cp = []
    for b in range(2):
        ck = pltpu.make_async_copy(
            k_hbm.at[b, pl.ds(row0, HQ_PER * DH), :], k_ref.at[b],
            cp_sems.at[b, 0],
        )
        cv = pltpu.make_async_copy(
            v_hbm.at[b, pl.ds(row0, HQ_PER * DH), :], v_ref.at[b],
            cp_sems.at[b, 1],
        )
        ck.start()
        cv.start()
        kv_cp.append((ck, cv))
    p1 = my ^ 1
    p2 = 3 - my

    sA = ((my + 1) >> 1) & 1
    qA = (my >> 1) & 1
    sB = (my >> 1) & 1
    qB = my & 1

    k1A = sA * 256
    s1A = (1 - sA) * 256
    k1B = 512 + sB * 256
    s1B = 512 + (1 - sB) * 256
    k2A = k1A + qA * 128
    s2A = k1A + (1 - qA) * 128
    k2B = k1B + qB * 128
    s2B = k1B + (1 - qB) * 128

    qA_p1 = (p1 >> 1) & 1
    p1_k2A = s1A + qA_p1 * 128
    p1_s2A = s1A + (1 - qA_p1) * 128
    qB_p2 = p2 & 1
    p2_k2B = s1B + qB_p2 * 128
    p2_s2B = s1B + (1 - qB_p2) * 128

    barrier = pltpu.get_barrier_semaphore()
    for nbr in (p1, p2):
        pl.semaphore_signal(
            barrier, inc=1, device_id=(nbr,), device_id_type=pl.DeviceIdType.MESH
        )
    pl.semaphore_wait(barrier, 2)

    def attn_core(ro, b_off, so_dyn, kv, nrows):
        b_idx = b_off // SQ
        xq = x_ref[pl.ds(ro, nrows), :]
        q = jnp.dot(xq, wq_ref[:, :], preferred_element_type=F32)
        q = q * 0.125
        rows = so_dyn + lax.broadcasted_iota(jnp.int32, (nrows, kv), 0)
        cols = lax.broadcasted_iota(jnp.int32, (nrows, kv), 1)
        mask = (cols // BLK) <= (rows // BLK)
        for h in range(HQ_PER):
            qh = q[:, h * DH:(h + 1) * DH]
            khT = k_ref[b_idx, pl.ds(h * DH, DH), pl.ds(0, kv)]
            s = lax.dot_general(
                qh, khT, (((1,), (0,)), ((), ())),
                preferred_element_type=F32,
            )
            e = jnp.exp(jnp.where(mask, s, -1e9))
            r = 1.0 / jnp.sum(e, axis=1, keepdims=True)
            vhT = v_ref[b_idx, pl.ds(h * DH, DH), pl.ds(0, kv)]
            c = lax.dot_general(
                e, vhT, (((1,), (1,)), ((), ())),
                preferred_element_type=F32,
            )
            ctx_scr[0:nrows, h * DH:(h + 1) * DH] = c * r
        out_ref[pl.ds(ro, nrows), :] = jnp.dot(
            ctx_scr[0:nrows, :], wo_ref[:, :], preferred_element_type=F32
        )

    def compute_rows(ro, nrows):
        so_dyn = ro % SQ
        b_off = ro - so_dyn

        @pl.when(so_dyn < 256)
        def _():
            attn_core(ro, b_off, so_dyn, 256, nrows)

        @pl.when(so_dyn >= 256)
        def _():
            attn_core(ro, b_off, so_dyn, SQ, nrows)

    def rdma(src, dst, sem, dev):
        return pltpu.make_async_remote_copy(
            src_ref=src, dst_ref=dst,
            send_sem=s_sems.at[sem], recv_sem=r_sems.at[sem],
            device_id=(dev,), device_id_type=pl.DeviceIdType.MESH,
        )

    kv_cp[0][0].wait()
    kv_cp[0][1].wait()
    compute_rows(s1A, 256)
    snd1[0, :, :] = out_ref[pl.ds(s1A, 256), :].astype(BF16)
    rs1A = rdma(snd1.at[0], rcv1.at[0], 0, p1)
    rs1A.start()
    kv_cp[1][0].wait()
    kv_cp[1][1].wait()
    compute_rows(s1B, 256)
    snd1[1, :, :] = out_ref[pl.ds(s1B, 256), :].astype(BF16)
    rs1B = rdma(snd1.at[1], rcv1.at[1], 1, p2)
    rs1B.start()
    compute_rows(k1A, 256)

    off_sA = (1 - qA) * 128
    off_kA = qA * 128
    rs1A.wait()
    out_ref[pl.ds(s2A, 128), :] += rcv1[0, pl.ds(off_sA, 128), :].astype(F32)
    snd2[0, :, :] = out_ref[pl.ds(s2A, 128), :].astype(BF16)
    rs2A = rdma(snd2.at[0], rcv2.at[0], 2, p2)
    rs2A.start()
    out_ref[pl.ds(k2A, 128), :] += rcv1[0, pl.ds(off_kA, 128), :].astype(F32)

    compute_rows(k1B, 256)

    off_sB = (1 - qB) * 128
    off_kB = qB * 128
    rs1B.wait()
    out_ref[pl.ds(s2B, 128), :] += rcv1[1, pl.ds(off_sB, 128), :].astype(F32)
    snd2[1, :, :] = out_ref[pl.ds(s2B, 128), :].astype(BF16)
    rs2B = rdma(snd2.at[1], rcv2.at[1], 3, p1)
    rs2B.start()
    out_ref[pl.ds(k2B, 128), :] += rcv1[1, pl.ds(off_kB, 128), :].astype(F32)

    rs2A.wait()
    out_ref[pl.ds(k2A, 128), :] += rcv2[0, :, :].astype(F32)
    sndq[0, :, :] = out_ref[pl.ds(k2A, 128), :].astype(BF16)
    ag1A = rdma(sndq.at[0], rcvq.at[0], 4, p2)
    ag1A.start()
    ag2A0 = rdma(sndq.at[0], rcvf.at[0, 0], 6, p1)
    ag2A0.start()

    rs2B.wait()
    out_ref[pl.ds(k2B, 128), :] += rcv2[1, :, :].astype(F32)
    sndq[1, :, :] = out_ref[pl.ds(k2B, 128), :].astype(BF16)
    ag1B = rdma(sndq.at[1], rcvq.at[1], 5, p1)
    ag1B.start()
    ag2B0 = rdma(sndq.at[1], rcvf.at[1, 0], 7, p2)
    ag2B0.start()

    ag1A.wait()
    out_ref[pl.ds(s2A, 128), :] = rcvq[0, :, :].astype(F32)
    ag2A1 = rdma(rcvq.at[0], rcvf.at[0, 1], 8, p1)
    ag2A1.start()
    ag1B.wait()
    out_ref[pl.ds(s2B, 128), :] = rcvq[1, :, :].astype(F32)
    ag2B1 = rdma(rcvq.at[1], rcvf.at[1, 1], 9, p2)
    ag2B1.start()

    ag2A0.wait()
    out_ref[pl.ds(p1_k2A, 128), :] = rcvf[0, 0, :, :].astype(F32)
    ag2A1.wait()
    out_ref[pl.ds(p1_s2A, 128), :] = rcvf[0, 1, :, :].astype(F32)
    ag2B0.wait()
    out_ref[pl.ds(p2_k2B, 128), :] = rcvf[1, 0, :, :].astype(F32)
    ag2B1.wait()
    out_ref[pl.ds(p2_s2B, 128), :] = rcvf[1, 1, :, :].astype(F32)


def kernel(x, Wq, K_ext, V_ext, Wo):
    B, Sq, d = x.shape
    my = lax.axis_index("i")

    K2 = jnp.transpose(K_ext, (0, 2, 3, 1)).reshape(B, 32 * DH, Sq)
    V2 = jnp.transpose(V_ext, (0, 2, 3, 1)).reshape(B, 32 * DH, Sq)
    x2 = x.reshape(B * Sq, d)

    out = pl.pallas_call(
        _body,
        out_shape=jax.ShapeDtypeStruct((B * Sq, d), jnp.float32),
        in_specs=[
            pl.BlockSpec(memory_space=pltpu.MemorySpace.VMEM),
            pl.BlockSpec(memory_space=pltpu.MemorySpace.VMEM),
            pl.BlockSpec(memory_space=pltpu.MemorySpace.HBM),
            pl.BlockSpec(memory_space=pltpu.MemorySpace.HBM),
            pl.BlockSpec(memory_space=pltpu.MemorySpace.VMEM),
        ],
        out_specs=pl.BlockSpec(memory_space=pltpu.MemorySpace.VMEM),
        scratch_shapes=[
            pltpu.VMEM((QROWS, HQ_PER * DH), F32),
            pltpu.VMEM((B, HQ_PER * DH, SQ), F32),
            pltpu.VMEM((B, HQ_PER * DH, SQ), F32),
            pltpu.VMEM((2, 256, D), BF16),
            pltpu.VMEM((2, 256, D), BF16),
            pltpu.VMEM((2, 128, D), BF16),
            pltpu.VMEM((2, 128, D), BF16),
            pltpu.VMEM((2, 128, D), BF16),
            pltpu.VMEM((2, 128, D), BF16),
            pltpu.VMEM((2, 2, 128, D), BF16),
            pltpu.SemaphoreType.DMA((10,)),
            pltpu.SemaphoreType.DMA((10,)),
            pltpu.SemaphoreType.DMA((2, 2)),
        ],
        compiler_params=pltpu.CompilerParams(collective_id=0),
    )(x2, Wq, K2, V2, Wo)
    return out.reshape(B, Sq, d)


# device time: 32506 ns/iter; 1.1395x vs baseline; 1.1395x over previous
import jax
import jax.numpy as jnp
from jax import lax
from jax.experimental import pallas as pl
from jax.experimental.pallas import tpu as pltpu

N_DEV = 4
HQ_PER = 8
DH = 64
BLK = 64
SQ = 512
QROWS = 256
D = 768

BF16 = jnp.bfloat16
F32 = jnp.float32


def _body(x_ref, wq_ref, k_ref, v_ref, wo_ref, out_ref, ctx_scr,
          bias0, bias1, snd1, rcv1, snd2, rcv2, sndq, rcvq, rcvf,
          s_sems, r_sems):
    my = lax.axis_index("i")

    r0 = lax.broadcasted_iota(jnp.int32, (QROWS, 256), 0) // BLK
    c0 = lax.broadcasted_iota(jnp.int32, (QROWS, 256), 1) // BLK
    bias0[:, :] = jnp.where(c0 <= r0, 0.0, -1e9)
    r1 = (256 + lax.broadcasted_iota(jnp.int32, (QROWS, SQ), 0)) // BLK
    c1 = lax.broadcasted_iota(jnp.int32, (QROWS, SQ), 1) // BLK
    bias1[:, :] = jnp.where(c1 <= r1, 0.0, -1e9)
    p1 = my ^ 1
    p2 = 3 - my

    sA = ((my + 1) >> 1) & 1
    qA = (my >> 1) & 1
    sB = (my >> 1) & 1
    qB = my & 1

    k1A = sA * 256
    s1A = (1 - sA) * 256
    k1B = 512 + sB * 256
    s1B = 512 + (1 - sB) * 256
    k2A = k1A + qA * 128
    s2A = k1A + (1 - qA) * 128
    k2B = k1B + qB * 128
    s2B = k1B + (1 - qB) * 128

    qA_p1 = (p1 >> 1) & 1
    p1_k2A = s1A + qA_p1 * 128
    p1_s2A = s1A + (1 - qA_p1) * 128
    qB_p2 = p2 & 1
    p2_k2B = s1B + qB_p2 * 128
    p2_s2B = s1B + (1 - qB_p2) * 128

    barrier = pltpu.get_barrier_semaphore()
    for nbr in (p1, p2):
        pl.semaphore_signal(
            barrier, inc=1, device_id=(nbr,), device_id_type=pl.DeviceIdType.MESH
        )
    pl.semaphore_wait(barrier, 2)

    def attn_core(ro, b_off, bias_ref, kv, nrows):
        b_idx = b_off // SQ
        xq = x_ref[pl.ds(ro, nrows), :]
        q = jnp.dot(xq, wq_ref[:, :], preferred_element_type=F32)
        q = q * 0.125
        bias = bias_ref[0:nrows, :]
        for h in range(HQ_PER):
            qh = q[:, h * DH:(h + 1) * DH]
            khT = k_ref[b_idx, pl.ds(h * DH, DH), pl.ds(0, kv)]
            s = lax.dot_general(
                qh, khT, (((1,), (0,)), ((), ())),
                preferred_element_type=F32,
            )
            e = jnp.exp(s + bias)
            r = 1.0 / jnp.sum(e, axis=1, keepdims=True)
            vhT = v_ref[b_idx, pl.ds(h * DH, DH), pl.ds(0, kv)]
            c = lax.dot_general(
                e, vhT, (((1,), (1,)), ((), ())),
                preferred_element_type=F32,
            )
            ctx_scr[0:nrows, h * DH:(h + 1) * DH] = c * r
        out_ref[pl.ds(ro, nrows), :] = jnp.dot(
            ctx_scr[0:nrows, :], wo_ref[:, :], preferred_element_type=F32
        )

    def compute_rows(ro, nrows):
        so_dyn = ro % SQ
        b_off = ro - so_dyn

        @pl.when(so_dyn < 256)
        def _():
            attn_core(ro, b_off, bias0, 256, nrows)

        @pl.when(so_dyn >= 256)
        def _():
            attn_core(ro, b_off, bias1, SQ, nrows)

    def rdma(src, dst, sem, dev):
        return pltpu.make_async_remote_copy(
            src_ref=src, dst_ref=dst,
            send_sem=s_sems.at[sem], recv_sem=r_sems.at[sem],
            device_id=(dev,), device_id_type=pl.DeviceIdType.MESH,
        )

    compute_rows(s1A, 256)
    snd1[0, :, :] = out_ref[pl.ds(s1A, 256), :].astype(BF16)
    rs1A = rdma(snd1.at[0], rcv1.at[0], 0, p1)
    rs1A.start()
    compute_rows(s1B, 256)
    snd1[1, :, :] = out_ref[pl.ds(s1B, 256), :].astype(BF16)
    rs1B = rdma(snd1.at[1], rcv1.at[1], 1, p2)
    rs1B.start()
    compute_rows(k1A, 256)

    off_sA = (1 - qA) * 128
    off_kA = qA * 128
    rs1A.wait()
    out_ref[pl.ds(s2A, 128), :] += rcv1[0, pl.ds(off_sA, 128), :].astype(F32)
    snd2[0, :, :] = out_ref[pl.ds(s2A, 128), :].astype(BF16)
    rs2A = rdma(snd2.at[0], rcv2.at[0], 2, p2)
    rs2A.start()
    out_ref[pl.ds(k2A, 128), :] += rcv1[0, pl.ds(off_kA, 128), :].astype(F32)

    compute_rows(k1B, 256)

    off_sB = (1 - qB) * 128
    off_kB = qB * 128
    rs1B.wait()
    out_ref[pl.ds(s2B, 128), :] += rcv1[1, pl.ds(off_sB, 128), :].astype(F32)
    snd2[1, :, :] = out_ref[pl.ds(s2B, 128), :].astype(BF16)
    rs2B = rdma(snd2.at[1], rcv2.at[1], 3, p1)
    rs2B.start()
    out_ref[pl.ds(k2B, 128), :] += rcv1[1, pl.ds(off_kB, 128), :].astype(F32)

    rs2A.wait()
    out_ref[pl.ds(k2A, 128), :] += rcv2[0, :, :].astype(F32)
    sndq[0, :, :] = out_ref[pl.ds(k2A, 128), :].astype(BF16)
    ag1A = rdma(sndq.at[0], rcvq.at[0], 4, p2)
    ag1A.start()
    ag2A0 = rdma(sndq.at[0], rcvf.at[0, 0], 6, p1)
    ag2A0.start()

    rs2B.wait()
    out_ref[pl.ds(k2B, 128), :] += rcv2[1, :, :].astype(F32)
    sndq[1, :, :] = out_ref[pl.ds(k2B, 128), :].astype(BF16)
    ag1B = rdma(sndq.at[1], rcvq.at[1], 5, p1)
    ag1B.start()
    ag2B0 = rdma(sndq.at[1], rcvf.at[1, 0], 7, p2)
    ag2B0.start()

    ag1A.wait()
    out_ref[pl.ds(s2A, 128), :] = rcvq[0, :, :].astype(F32)
    ag2A1 = rdma(rcvq.at[0], rcvf.at[0, 1], 8, p1)
    ag2A1.start()
    ag1B.wait()
    out_ref[pl.ds(s2B, 128), :] = rcvq[1, :, :].astype(F32)
    ag2B1 = rdma(rcvq.at[1], rcvf.at[1, 1], 9, p2)
    ag2B1.start()

    ag2A0.wait()
    out_ref[pl.ds(p1_k2A, 128), :] = rcvf[0, 0, :, :].astype(F32)
    ag2A1.wait()
    out_ref[pl.ds(p1_s2A, 128), :] = rcvf[0, 1, :, :].astype(F32)
    ag2B0.wait()
    out_ref[pl.ds(p2_k2B, 128), :] = rcvf[1, 0, :, :].astype(F32)
    ag2B1.wait()
    out_ref[pl.ds(p2_s2B, 128), :] = rcvf[1, 1, :, :].astype(F32)


def kernel(x, Wq, K_ext, V_ext, Wo):
    B, Sq, d = x.shape
    my = lax.axis_index("i")

    Kt = jnp.transpose(K_ext, (0, 2, 3, 1))
    Vt = jnp.transpose(V_ext, (0, 2, 3, 1))
    K2 = lax.dynamic_slice_in_dim(Kt, my * HQ_PER, HQ_PER, axis=1)
    V2 = lax.dynamic_slice_in_dim(Vt, my * HQ_PER, HQ_PER, axis=1)
    K2 = K2.reshape(B, HQ_PER * DH, Sq)
    V2 = V2.reshape(B, HQ_PER * DH, Sq)
    x2 = x.reshape(B * Sq, d)

    out = pl.pallas_call(
        _body,
        out_shape=jax.ShapeDtypeStruct((B * Sq, d), jnp.float32),
        in_specs=[pl.BlockSpec(memory_space=pltpu.VMEM)] * 5,
        out_specs=pl.BlockSpec(memory_space=pltpu.VMEM),
        scratch_shapes=[
            pltpu.VMEM((QROWS, HQ_PER * DH), F32),
            pltpu.VMEM((QROWS, 256), F32),
            pltpu.VMEM((QROWS, SQ), F32),
            pltpu.VMEM((2, 256, D), BF16),
            pltpu.VMEM((2, 256, D), BF16),
            pltpu.VMEM((2, 128, D), BF16),
            pltpu.VMEM((2, 128, D), BF16),
            pltpu.VMEM((2, 128, D), BF16),
            pltpu.VMEM((2, 128, D), BF16),
            pltpu.VMEM((2, 2, 128, D), BF16),
            pltpu.SemaphoreType.DMA((10,)),
            pltpu.SemaphoreType.DMA((10,)),
        ],
        compiler_params=pltpu.CompilerParams(collective_id=0),
    )(x2, Wq, K2, V2, Wo)
    return out.reshape(B, Sq, d)
